# Initial kernel scaffold; baseline (speedup 1.0000x reference)
import jax
import jax.numpy as jnp
from jax.experimental import pallas as pl


def kernel(inputs, edge_index, W1, b1, W2, b2, epoch):
    raise NotImplementedError("write your pallas kernel here")


# speedup vs baseline: 9.1560x; 9.1560x over previous
import functools

import jax
import jax.numpy as jnp
from jax import lax
from jax.experimental import pallas as pl
from jax.experimental.pallas import tpu as pltpu
from jax.experimental.pallas import tpu_sc as plsc

N = 10000
E = 320000
D1 = 128
DC = 40
D2 = 128

NC = 2
NS = 16
NW = NC * NS
CHUNK = 128

NPAD = 10112
RPT = NPAD // NS

NCH = -(-E // (NW * CHUNK))
EPW = NCH * CHUNK
EPAD = NW * EPW

_MESH = plsc.VectorSubcoreMesh(core_axis_name="c", subcore_axis_name="s")


def _deg_body(dst_hbm, out_hbm, didx_v, deg_v):
    c = lax.axis_index("c")
    s = lax.axis_index("s")
    wid = s * NC + c

    zero16 = jnp.zeros((16,), jnp.float32)
    ones = jnp.ones((16,), jnp.float32)

    def zb(i, carry):
        deg_v[pl.ds(i * 16, 16)] = zero16
        return carry

    lax.fori_loop(0, NPAD // 16, zb, 0)

    def chunk(j, carry):
        base = wid * EPW + j * CHUNK
        pltpu.sync_copy(dst_hbm.at[pl.ds(base, CHUNK)], didx_v)
        for k in range(CHUNK // 16):
            idx16 = didx_v[pl.ds(k * 16, 16)]
            plsc.addupdate_scatter(deg_v, [idx16], ones)
        return carry

    lax.fori_loop(0, NCH, chunk, 0)
    pltpu.sync_copy(deg_v, out_hbm.at[wid, 0])


_deg_kernel = pl.kernel(
    _deg_body,
    out_type=jax.ShapeDtypeStruct((NW, 8, NPAD), jnp.float32),
    mesh=_MESH,
    scratch_types=[
        pltpu.VMEM((CHUNK,), jnp.int32),
        pltpu.VMEM((NPAD,), jnp.float32),
    ],
    compiler_params=pltpu.CompilerParams(needs_layout_passes=False),
)


def _make_prop(d):
    def body(g_hbm, src_hbm, dst_hbm, z_hbm, out_hbm,
             sidx_v, didx_v, rows_v, acc, sem):
        c = lax.axis_index("c")
        s = lax.axis_index("s")
        wid = s * NC + c
        r0 = s * RPT

        def row_chunks(fn):
            off = 0
            while off < RPT:
                cb = min(CHUNK, RPT - off)
                fn(off, cb)
                off += cb

        def init(off, cb):
            pltpu.sync_copy(z_hbm.at[pl.ds(r0 + off, cb)],
                            rows_v.at[pl.ds(0, cb)])
            pltpu.sync_copy(rows_v.at[pl.ds(0, cb)],
                            acc.at[pl.ds(r0 + off, cb)])

        row_chunks(init)
        plsc.subcore_barrier()

        def chunk(j, carry):
            base = wid * EPW + j * CHUNK
            pltpu.sync_copy(src_hbm.at[pl.ds(base, CHUNK)], sidx_v)
            pltpu.sync_copy(dst_hbm.at[pl.ds(base, CHUNK)], didx_v)
            pltpu.async_copy(g_hbm.at[sidx_v], rows_v, sem).wait()
            pltpu.sync_copy(rows_v, acc.at[didx_v], add=True)
            return carry

        lax.fori_loop(0, NCH, chunk, 0)
        plsc.subcore_barrier()

        def writeback(off, cb):
            pltpu.sync_copy(acc.at[pl.ds(r0 + off, cb)],
                            rows_v.at[pl.ds(0, cb)])
            pltpu.sync_copy(rows_v.at[pl.ds(0, cb)],
                            out_hbm.at[c, pl.ds(r0 + off, cb)])

        row_chunks(writeback)

    return pl.kernel(
        body,
        out_type=jax.ShapeDtypeStruct((NC, NPAD, d), jnp.float32),
        mesh=_MESH,
        scratch_types=[
            pltpu.VMEM((CHUNK,), jnp.int32),
            pltpu.VMEM((CHUNK,), jnp.int32),
            pltpu.VMEM((CHUNK, d), jnp.float32),
            pltpu.VMEM_SHARED((NPAD, d), jnp.float32),
            pltpu.SemaphoreType.DMA,
        ],
    )


_prop128 = _make_prop(D1)


def _stage_a_body(x_ref, w1_ref, degp_ref, g_ref, dinv_ref):
    deg0 = jnp.sum(degp_ref[...], axis=1, keepdims=True)
    dinv = lax.rsqrt(jnp.maximum(deg0, 1.0))
    dinv_ref[...] = dinv
    h = jnp.dot(x_ref[...], w1_ref[...], preferred_element_type=jnp.float32)
    g_ref[...] = h * dinv


_stage_a = pl.pallas_call(
    _stage_a_body,
    out_shape=[
        jax.ShapeDtypeStruct((NPAD, D1), jnp.float32),
        jax.ShapeDtypeStruct((NPAD, 1), jnp.float32),
    ],
)


def _stage_b_body(p_ref, dinv_ref, b1_ref, w2_ref, g2_ref):
    acc = p_ref[0] + p_ref[1]
    dinv = dinv_ref[...]
    h = jnp.maximum(acc * dinv + b1_ref[...], 0.0)
    g2_ref[...] = jnp.dot(h, w2_ref[...],
                          preferred_element_type=jnp.float32) * dinv


_stage_b = pl.pallas_call(
    _stage_b_body,
    out_shape=jax.ShapeDtypeStruct((NPAD, D2), jnp.float32),
)


def _stage_c_body(q_ref, dinv_ref, b2_ref, o_ref):
    acc = q_ref[0] + q_ref[1]
    logits = acc * dinv_ref[...] + b2_ref[...]
    col = lax.broadcasted_iota(jnp.int32, (NPAD, D2), 1)
    valid = col < DC
    logits = jnp.where(valid, logits, -jnp.inf)
    m = jnp.max(logits, axis=1, keepdims=True)
    ex = jnp.where(valid, jnp.exp(logits - m), 0.0)
    lse = jnp.log(jnp.sum(ex, axis=1, keepdims=True))
    out = logits - m - lse
    o_ref[...] = out[:N, :DC]


_stage_c = pl.pallas_call(
    _stage_c_body,
    out_shape=jax.ShapeDtypeStruct((N, DC), jnp.float32),
)


def kernel(inputs, edge_index, W1, b1, W2, b2, epoch):
    ei = edge_index.astype(jnp.int32)
    pad = jnp.full((EPAD - E,), N, dtype=jnp.int32)
    src = jnp.concatenate([ei[0], pad])
    dst = jnp.concatenate([ei[1], pad])

    x = jnp.concatenate(
        [inputs, jnp.zeros((NPAD - N, D1), jnp.float32)], axis=0)
    w2p = jnp.concatenate(
        [W2, jnp.zeros((D1, D2 - DC), jnp.float32)], axis=1)
    b1r = b1.reshape(1, D1)
    b2r = jnp.concatenate([b2, jnp.zeros((D2 - DC,), jnp.float32)]
                          ).reshape(1, D2)

    z128 = jnp.zeros((NPAD, D1), jnp.float32)

    degp = _deg_kernel(dst)
    degt = jnp.transpose(degp[:, 0, :])
    g1, dinv = _stage_a(x, W1, degt)
    p = _prop128(g1, src, dst, z128)
    g2 = _stage_b(p, dinv, b1r, w2p)
    q = _prop128(g2, src, dst, z128)
    return _stage_c(q, dinv, b2r)
